# baseline (speedup 1.0000x reference)
import functools
import math

import jax
import jax.numpy as jnp
from jax import lax
from jax.experimental import pallas as pl
from jax.experimental.pallas import tpu as pltpu
from jax.experimental.pallas import tpu_sc as plsc

_NC = 2
_NS = 16
_NW = _NC * _NS
_LANES = 16
_PAD = 129


def _emb_body(idxT_hbm, tbl_hbm, out_hbm, idxall, rows0, rows1, tile0,
              tile1, tight0, tight1, isem, gsem, ssem, *, nj, tpw, d,
              scale):
    wid = lax.axis_index("s") * _NC + lax.axis_index("c")
    colbase = wid * tpw
    ntb = tpw // 128
    tb0 = wid * ntb
    rows = (rows0, rows1)
    tiles = (tile0, tile1)
    tights = (tight0, tight1)

    for j in range(nj):
        pltpu.async_copy(idxT_hbm.at[j, pl.ds(colbase, tpw)],
                         idxall.at[j], isem)

    def idx_wait(j):
        pltpu.make_async_copy(idxT_hbm.at[j, pl.ds(colbase, tpw)],
                              idxall.at[j], isem).wait()

    def gather(j, b):
        return pltpu.async_copy(tbl_hbm.at[idxall.at[j]], rows[b], gsem)

    def stores(j, b, start):
        for tbl in range(ntb):
            for tc in range(d // 8):
                c = pltpu.make_async_copy(
                    tights[b].at[tbl, tc],
                    out_hbm.at[j, tc, tb0 + tbl], ssem)
                if start:
                    c.start()
                else:
                    c.wait()

    def compact(b):
        def cpk(k, c2, _b=b):
            tbl = k // (d // 8)
            tc = k - tbl * (d // 8)
            for cm in range(8):
                for g in range(8):
                    tights[_b][tbl, tc, cm, pl.ds(g * _LANES, _LANES)] = (
                        tiles[_b][tbl, tc, cm, pl.ds(g * _LANES, _LANES)])
            return c2

        lax.fori_loop(0, ntb * (d // 8), cpk, 0)

    idx_wait(0)
    gather(0, 0)
    iota = lax.iota(jnp.int32, _LANES)
    tc_a = iota // 8
    cm_a = iota % 8
    tc_b = tc_a + 2
    zeros = jnp.zeros((_LANES,), jnp.int32)

    def pair(p, carry):
        for b in (0, 1):
            j = p * 2 + b

            @pl.when(j >= 2)
            def _drain():
                stores(j - 2, b, start=False)

            @pl.when(j + 1 < nj)
            def _next():
                idx_wait(j + 1)
                gather(j + 1, 1 - b)

            pltpu.make_async_copy(tbl_hbm.at[idxall.at[j]], rows[b],
                                  gsem).wait()

            for tbl in range(ntb):
                tbase = tbl * 128
                tbl_v = zeros + tbl

                def tbody(k, c2, _b=b, _tbase=tbase, _tbl_v=tbl_v):
                    for u in range(4):
                        im = k * 4 + u
                        t = _tbase + im
                        va = rows[_b][t, pl.ds(0, _LANES)] * scale
                        vb = rows[_b][t, pl.ds(_LANES, _LANES)] * scale
                        im_v = zeros + im
                        plsc.store_scatter(tiles[_b],
                                           [_tbl_v, tc_a, cm_a, im_v], va)
                        plsc.store_scatter(tiles[_b],
                                           [_tbl_v, tc_b, cm_a, im_v], vb)
                    return c2

                lax.fori_loop(0, 32, tbody, 0)
            compact(b)
            stores(j, b, start=True)
        return carry

    lax.fori_loop(0, nj // 2, pair, 0)
    stores(nj - 2, 0, start=False)
    stores(nj - 1, 1, start=False)


def kernel(input, weight, sigma, length_scale):
    n_tok, nj = input.shape
    d = weight.shape[1]
    scale = math.sqrt(d)
    tpw = n_tok // _NW
    idxT = jnp.swapaxes(input, 0, 1).astype(jnp.int32)

    mesh = plsc.VectorSubcoreMesh(core_axis_name="c", subcore_axis_name="s",
                                  num_cores=_NC, num_subcores=_NS)
    body = functools.partial(_emb_body, nj=nj, tpw=tpw, d=d, scale=scale)
    ntb = tpw // 128
    oT5 = pl.kernel(
        body,
        out_type=jax.ShapeDtypeStruct((nj, d // 8, n_tok // 128, 8, 128),
                                      jnp.float32),
        mesh=mesh,
        compiler_params=pltpu.CompilerParams(use_tc_tiling_on_sc=False,
                                             needs_layout_passes=False),
        scratch_types=[
            pltpu.VMEM((nj, tpw), jnp.int32),
            pltpu.VMEM((tpw, d), jnp.float32),
            pltpu.VMEM((tpw, d), jnp.float32),
            pltpu.VMEM((ntb, d // 8, 8, _PAD), jnp.float32),
            pltpu.VMEM((ntb, d // 8, 8, _PAD), jnp.float32),
            pltpu.VMEM((ntb, d // 8, 8, 128), jnp.float32),
            pltpu.VMEM((ntb, d // 8, 8, 128), jnp.float32),
            pltpu.SemaphoreType.DMA,
            pltpu.SemaphoreType.DMA,
            pltpu.SemaphoreType.DMA,
        ],
    )(idxT, weight)
    return oT5.transpose(2, 4, 0, 1, 3).reshape(n_tok, nj, d)


# speedup vs baseline: 1.4109x; 1.4109x over previous
import functools
import math

import jax
import jax.numpy as jnp
from jax import lax
from jax.experimental import pallas as pl
from jax.experimental.pallas import tpu as pltpu
from jax.experimental.pallas import tpu_sc as plsc

_NC = 2
_NS = 16
_NW = _NC * _NS
_LANES = 16
_PAD = 129


def _emb_body(idxT_hbm, tbl_hbm, out_hbm, idxall, rows0, rows1, tile0,
              tile1, isem, gsem, ssem, *, nj, tpw, d, scale):
    wid = lax.axis_index("s") * _NC + lax.axis_index("c")
    colbase = wid * tpw
    ntb = tpw // 128
    tb0 = wid * ntb
    rows = (rows0, rows1)
    tiles = (tile0, tile1)

    for j in range(nj):
        pltpu.async_copy(idxT_hbm.at[j, pl.ds(colbase, tpw)],
                         idxall.at[j], isem)

    def idx_wait(j):
        pltpu.make_async_copy(idxT_hbm.at[j, pl.ds(colbase, tpw)],
                              idxall.at[j], isem).wait()

    def gather(j, b):
        return pltpu.async_copy(tbl_hbm.at[idxall.at[j]], rows[b], gsem)

    def stores(j, b, start):
        for tbl in range(ntb):
            for tc in range(d // 8):
                c = pltpu.make_async_copy(
                    tiles[b].at[tbl, tc, slice(None), pl.ds(0, 128)],
                    out_hbm.at[j, tc, tb0 + tbl], ssem)
                if start:
                    c.start()
                else:
                    c.wait()


    idx_wait(0)
    gather(0, 0)
    iota = lax.iota(jnp.int32, _LANES)
    tc_a = iota // 8
    cm_a = iota % 8
    tc_b = tc_a + 2
    zeros = jnp.zeros((_LANES,), jnp.int32)

    def pair(p, carry):
        for b in (0, 1):
            j = p * 2 + b

            @pl.when(j >= 2)
            def _drain():
                stores(j - 2, b, start=False)

            @pl.when(j + 1 < nj)
            def _next():
                idx_wait(j + 1)
                gather(j + 1, 1 - b)

            pltpu.make_async_copy(tbl_hbm.at[idxall.at[j]], rows[b],
                                  gsem).wait()

            for tbl in range(ntb):
                tbase = tbl * 128
                tbl_v = zeros + tbl

                def run_tloop(_b=b, _tbase=tbase, _tbl_v=tbl_v):
                    @plsc.parallel_loop(0, 128, 1, unroll=4)
                    def _tloop(im):
                        t = _tbase + im
                        va = rows[_b][t, pl.ds(0, _LANES)] * scale
                        vb = rows[_b][t, pl.ds(_LANES, _LANES)] * scale
                        im_v = zeros + im
                        plsc.store_scatter(tiles[_b],
                                           [_tbl_v, tc_a, cm_a, im_v], va)
                        plsc.store_scatter(tiles[_b],
                                           [_tbl_v, tc_b, cm_a, im_v], vb)

                run_tloop()
            stores(j, b, start=True)
        return carry

    lax.fori_loop(0, nj // 2, pair, 0)
    stores(nj - 2, 0, start=False)
    stores(nj - 1, 1, start=False)


def kernel(input, weight, sigma, length_scale):
    n_tok, nj = input.shape
    d = weight.shape[1]
    scale = math.sqrt(d)
    tpw = n_tok // _NW
    idxT = jnp.swapaxes(input, 0, 1).astype(jnp.int32)

    mesh = plsc.VectorSubcoreMesh(core_axis_name="c", subcore_axis_name="s",
                                  num_cores=_NC, num_subcores=_NS)
    body = functools.partial(_emb_body, nj=nj, tpw=tpw, d=d, scale=scale)
    ntb = tpw // 128
    oT5 = pl.kernel(
        body,
        out_type=jax.ShapeDtypeStruct((nj, d // 8, n_tok // 128, 8, 128),
                                      jnp.float32),
        mesh=mesh,
        compiler_params=pltpu.CompilerParams(use_tc_tiling_on_sc=False,
                                             needs_layout_passes=False),
        scratch_types=[
            pltpu.VMEM((nj, tpw), jnp.int32),
            pltpu.VMEM((tpw, d), jnp.float32),
            pltpu.VMEM((tpw, d), jnp.float32),
            pltpu.VMEM((ntb, d // 8, 8, _PAD), jnp.float32),
            pltpu.VMEM((ntb, d // 8, 8, _PAD), jnp.float32),
            pltpu.SemaphoreType.DMA,
            pltpu.SemaphoreType.DMA,
            pltpu.SemaphoreType.DMA,
        ],
    )(idxT, weight)
    return oT5.transpose(2, 4, 0, 1, 3).reshape(n_tok, nj, d)
